# baseline (speedup 1.0000x reference)
import functools

import jax
import jax.numpy as jnp
from jax.experimental import pallas as pl


def _edge_kernel(u_ref, *refs, blk_e, quarter, n_stream):
    w_refs = refs[:n_stream]
    b_refs = refs[n_stream:2 * n_stream]
    s_ref = refs[2 * n_stream]
    y_refs = refs[2 * n_stream + 1:]
    i = pl.program_id(0)
    u = u_ref[...]
    ub = u.astype(jnp.bfloat16)
    s = s_ref[0, 0]
    for k in range(n_stream):
        z = jax.lax.dot_general(
            ub, w_refs[k][...].astype(jnp.bfloat16),
            (((1,), (1,)), ((), ())),
            preferred_element_type=jnp.float32,
        )
        dyn = jax.nn.sigmoid(z + b_refs[k][0, :][None, :])
        u_blk = u_ref[:, pl.ds(k * quarter + i * blk_e, blk_e)]
        y_refs[k][...] = s * u_blk + (1.0 - s) * dyn


def kernel(hidden_states, edge_index, edge_weight, W, b, skip_param):
    seq_len = hidden_states.shape[1]
    E = W.shape[0]
    BE = edge_weight.shape[0]
    batch = BE // E
    half = E // 2

    u = edge_weight.reshape(batch, E)
    b2 = b.reshape(1, E)
    s2 = skip_param.reshape(1, 1)

    n_stream = 4
    quarter = E // n_stream
    blk_e = 128
    n_blk = quarter // blk_e

    body = functools.partial(_edge_kernel, blk_e=blk_e, quarter=quarter,
                             n_stream=n_stream)

    w_specs = [
        pl.BlockSpec((blk_e, E), functools.partial(
            lambda k, i: (i + k * n_blk, 0), k))
        for k in range(n_stream)
    ]
    b_specs = [
        pl.BlockSpec((1, blk_e), functools.partial(
            lambda k, i: (0, i + k * n_blk), k))
        for k in range(n_stream)
    ]

    ys = pl.pallas_call(
        body,
        grid=(n_blk,),
        in_specs=(
            [pl.BlockSpec((batch, E), lambda i: (0, 0))]
            + w_specs + b_specs
            + [pl.BlockSpec((1, 1), lambda i: (0, 0))]
        ),
        out_specs=[pl.BlockSpec((batch, blk_e), lambda i: (0, i))
                   for _ in range(n_stream)],
        out_shape=[jax.ShapeDtypeStruct((batch, quarter), jnp.float32)
                   for _ in range(n_stream)],
    )(u, *([W] * n_stream), *([b2] * n_stream), s2)

    y2 = jnp.concatenate(ys, axis=1)
    ei3 = jnp.broadcast_to(edge_index[:, :, None], (2, BE, seq_len))
    out = jnp.broadcast_to(y2.reshape(BE, 1), (BE, seq_len))
    return ei3, out
